# baseline (speedup 1.0000x reference)
import functools
import math

import jax
import jax.numpy as jnp
import numpy as np
from jax import lax
from jax.experimental import pallas as pl
from jax.experimental.pallas import tpu as pltpu
from jax.experimental.pallas import tpu_sc as plsc

G = 12
D = 128
T = 50
B = 4096
TEMP = 1.0
AUX_W = 5.0
ID_ID = 0
NW = 32
BPW = B // NW
LANES = 16


def _dot(a, b):
    return lax.dot_general(a, b, (((1,), (0,)), ((), ())),
                           preferred_element_type=jnp.float32,
                           precision=lax.Precision.HIGHEST)


_CB = np.zeros((G, 16), np.float64)
for _g in range(G):
    for _f in range(1, 6):
        _CB[_g, _f - 1] = math.cos(2 * math.pi * _g * _f / G)
        _CB[_g, 4 + _f] = -math.sin(2 * math.pi * _g * _f / G)
    _CB[_g, 10] = (-1.0) ** _g
    _CB[_g, 11] = 1.0
_CBASIS = _CB.astype(np.float32)

_E1T = np.repeat(np.eye(G, dtype=np.float32), G, axis=0).T.copy()
_E2T = np.tile(np.eye(G, dtype=np.float32), (G, 1)).T.copy()
_RSEL = np.zeros((G, 256), np.float32)
for _v in range(G):
    _RSEL[_v, _v * LANES:(_v + 1) * LANES] = 1.0


def _dot0(a, b):
    return lax.dot_general(a, b, (((0,), (0,)), ((), ())),
                           preferred_element_type=jnp.float32,
                           precision=lax.Precision.HIGHEST)


def _table_body(embed_ref, W1_ref, b1_ref, W2_ref, b2_ref, C_ref,
                E1T_ref, E2T_ref, L_ref, Qp_ref):
    z = jnp.maximum(_dot(embed_ref[...], W1_ref[...]) + b1_ref[...], 0.0)
    l = _dot(z, W2_ref[...]) + b2_ref[...]
    L_ref[...] = l
    m = jnp.max(l, axis=1, keepdims=True)
    e = jnp.exp((l - m) * (1.0 / TEMP))
    p = e / jnp.sum(e, axis=1, keepdims=True)
    qT = lax.dot_general(C_ref[...], p, (((0,), (1,)), ((), ())),
                         preferred_element_type=jnp.float32,
                         precision=lax.Precision.HIGHEST)
    qa = _dot(qT, E1T_ref[...])
    qb = _dot(qT, E2T_ref[...])
    are, aim = qa[0:5, :], qa[5:10, :]
    bre, bim = qb[0:5, :], qb[5:10, :]
    body = jnp.concatenate([
        are * bre - aim * bim,
        are * bim + aim * bre,
        qa[10:11, :] * qb[10:11, :],
        qa[11:12, :] * qb[11:12, :],
        jnp.zeros((4, G * G), jnp.float32),
    ], axis=0)
    Qp_ref[...] = jnp.concatenate(
        [body, jnp.zeros((16, 256 - G * G), jnp.float32)], axis=1)


def _final_body(s_ref, lab_ref, cnt_ref, L_ref, RSEL_ref,
                logits_ref, loss_ref):
    s = jnp.maximum(s_ref[0:G, :], 1e-9)
    logits = jnp.log(s)
    logits_ref[...] = logits
    lab = lab_ref[...]
    onehot = (lax.broadcasted_iota(jnp.int32, (G, B), 0) == lab
              ).astype(jnp.float32)
    lse = jnp.log(jnp.sum(s, axis=0, keepdims=True))
    picked = jnp.sum(onehot * logits, axis=0, keepdims=True)
    loss_final = jnp.sum(lse - picked).reshape(1, 1) * (1.0 / B)
    L = L_ref[...]
    Lm = jnp.max(L, axis=1, keepdims=True)
    Llse = jnp.log(jnp.sum(jnp.exp(L - Lm), axis=1, keepdims=True)) + Lm
    diag = (lax.broadcasted_iota(jnp.int32, (G, G), 0) ==
            lax.broadcasted_iota(jnp.int32, (G, G), 1)).astype(jnp.float32)
    Ldiag = jnp.sum(L * diag, axis=1, keepdims=True)
    r = Llse - Ldiag
    rexp = _dot0(r, RSEL_ref[...])
    total = jnp.sum(cnt_ref[...], axis=0, keepdims=True)
    route_sum = jnp.sum(total * rexp).reshape(1, 1)
    loss_ref[...] = loss_final + AUX_W * route_sum * (1.0 / (B * T))


def _sc_scan_body(ids_hbm, Q_hbm, out_hbm, cnt_hbm, ids_v, Q_v, out_v, cnt_v):
    wid = lax.axis_index("s") * 2 + lax.axis_index("c")
    base = wid * BPW
    pltpu.sync_copy(ids_hbm.at[pl.ds(base, BPW), :], ids_v)
    pltpu.sync_copy(Q_hbm, Q_v)
    zeros = jnp.zeros((LANES,), jnp.float32)
    ones = jnp.full((LANES,), 1.0, jnp.float32)
    lane = lax.iota(jnp.int32, LANES)
    for v in range(256 // LANES):
        cnt_v[pl.ds(v * LANES, LANES)] = zeros
    for j in range(G, 16):
        for c in range(BPW // LANES):
            out_v[j, pl.ds(c * LANES, LANES)] = zeros
    for c in range(BPW // LANES):
        col = c * LANES
        init = (ones,) * 6 + (zeros,) * 5 + (ones,)

        rowvec = col + lane

        def step(t, S, col=col, rowvec=rowvec):
            ta = jnp.full((LANES,), 2 * t, jnp.int32)
            ia = plsc.load_gather(ids_v, [rowvec, ta])
            ib = plsc.load_gather(ids_v, [rowvec, ta + 1])
            a16 = ia * 16
            b16 = ib * 16
            plsc.addupdate_scatter(cnt_v, [a16 + lane], ones)
            plsc.addupdate_scatter(cnt_v, [b16 + lane], ones)
            pidx = ia * G + ib
            q = [plsc.load_gather(Q_v, [pidx + 256 * qq]) for qq in range(12)]
            S0, Sre, Sim, S6 = S[0], S[1:6], S[6:11], S[11]
            nre, nim = [], []
            for f in range(5):
                nre.append(Sre[f] * q[f] - Sim[f] * q[5 + f])
                nim.append(Sre[f] * q[5 + f] + Sim[f] * q[f])
            return (S0 * q[11],) + tuple(nre) + tuple(nim) + (S6 * q[10],)

        S = lax.fori_loop(0, T // 2, step, init)
        S0, Sre, Sim, S6 = S[0], S[1:6], S[6:11], S[11]
        inv = 1.0 / G
        for j in range(G):
            acc = S0 * inv + S6 * (((-1.0) ** j) * inv)
            for f in range(1, 6):
                cA = 2.0 * math.cos(2 * math.pi * j * f / G) * inv
                cB = -2.0 * math.sin(2 * math.pi * j * f / G) * inv
                if abs(cA) > 1e-9:
                    acc = acc + Sre[f - 1] * cA
                if abs(cB) > 1e-9:
                    acc = acc + Sim[f - 1] * cB
            out_v[j, pl.ds(col, LANES)] = acc
    pltpu.sync_copy(out_v, out_hbm.at[:, pl.ds(base, BPW)])
    pltpu.sync_copy(cnt_v, cnt_hbm.at[wid])


@functools.cache
def _sc_scan_kernel():
    return pl.kernel(
        _sc_scan_body,
        out_type=[
            jax.ShapeDtypeStruct((16, B), jnp.float32),
            jax.ShapeDtypeStruct((NW, 256), jnp.float32),
        ],
        mesh=plsc.VectorSubcoreMesh(core_axis_name="c", subcore_axis_name="s",
                                    num_cores=2, num_subcores=16),
        scratch_types=[
            pltpu.VMEM((BPW, T), jnp.int32),
            pltpu.VMEM((16 * 256,), jnp.float32),
            pltpu.VMEM((16, BPW), jnp.float32),
            pltpu.VMEM((256,), jnp.float32),
        ],
        compiler_params=pltpu.CompilerParams(needs_layout_passes=False),
    )


def kernel(input_ids, labels, mul, embed, W1, b1, W2, b2):
    del mul
    L, Qp = pl.pallas_call(
        _table_body,
        out_shape=[
            jax.ShapeDtypeStruct((G, G), jnp.float32),
            jax.ShapeDtypeStruct((16, 256), jnp.float32),
        ],
    )(embed, W1, b1, W2, b2,
      jnp.asarray(_CBASIS), jnp.asarray(_E1T), jnp.asarray(_E2T))

    s_final16, cnt = _sc_scan_kernel()(input_ids, Qp.reshape(16 * 256))

    logitsT, loss = pl.pallas_call(
        _final_body,
        out_shape=[
            jax.ShapeDtypeStruct((G, B), jnp.float32),
            jax.ShapeDtypeStruct((1, 1), jnp.float32),
        ],
    )(s_final16, labels, cnt, L, jnp.asarray(_RSEL))
    return (logitsT.T, loss.reshape(()))


# speedup vs baseline: 1.1827x; 1.1827x over previous
import functools
import math

import jax
import jax.numpy as jnp
import numpy as np
from jax import lax
from jax.experimental import pallas as pl
from jax.experimental.pallas import tpu as pltpu
from jax.experimental.pallas import tpu_sc as plsc

G = 12
D = 128
T = 50
B = 4096
TEMP = 1.0
AUX_W = 5.0
ID_ID = 0
NW = 32
BPW = B // NW
LANES = 16


def _dot(a, b):
    return lax.dot_general(a, b, (((1,), (0,)), ((), ())),
                           preferred_element_type=jnp.float32,
                           precision=lax.Precision.HIGHEST)


_CB = np.zeros((G, 16), np.float64)
for _g in range(G):
    for _f in range(1, 6):
        _CB[_g, _f - 1] = math.cos(2 * math.pi * _g * _f / G)
        _CB[_g, 4 + _f] = -math.sin(2 * math.pi * _g * _f / G)
    _CB[_g, 10] = (-1.0) ** _g
    _CB[_g, 11] = 1.0
_CBASIS = _CB.astype(np.float32)

_E1T = np.repeat(np.eye(G, dtype=np.float32), G, axis=0).T.copy()
_E2T = np.tile(np.eye(G, dtype=np.float32), (G, 1)).T.copy()
_RSEL = np.zeros((G, 256), np.float32)
for _v in range(G):
    _RSEL[_v, _v * LANES:(_v + 1) * LANES] = 1.0


def _dot0(a, b):
    return lax.dot_general(a, b, (((0,), (0,)), ((), ())),
                           preferred_element_type=jnp.float32,
                           precision=lax.Precision.HIGHEST)


def _table_body(embed_ref, W1_ref, b1_ref, W2_ref, b2_ref, C_ref,
                E1T_ref, E2T_ref, L_ref, Qp_ref):
    z = jnp.maximum(_dot(embed_ref[...], W1_ref[...]) + b1_ref[...], 0.0)
    l = _dot(z, W2_ref[...]) + b2_ref[...]
    L_ref[...] = l
    m = jnp.max(l, axis=1, keepdims=True)
    e = jnp.exp((l - m) * (1.0 / TEMP))
    p = e / jnp.sum(e, axis=1, keepdims=True)
    qT = lax.dot_general(C_ref[...], p, (((0,), (1,)), ((), ())),
                         preferred_element_type=jnp.float32,
                         precision=lax.Precision.HIGHEST)
    qa = _dot(qT, E1T_ref[...])
    qb = _dot(qT, E2T_ref[...])
    are, aim = qa[0:5, :], qa[5:10, :]
    bre, bim = qb[0:5, :], qb[5:10, :]
    body = jnp.concatenate([
        are * bre - aim * bim,
        are * bim + aim * bre,
        qa[10:11, :] * qb[10:11, :],
        qa[11:12, :] * qb[11:12, :],
        jnp.zeros((4, G * G), jnp.float32),
    ], axis=0)
    Qp_ref[...] = jnp.concatenate(
        [body, jnp.zeros((16, 256 - G * G), jnp.float32)], axis=1)


def _final_body(s_ref, lab_ref, cnt_ref, L_ref, RSEL_ref,
                logits_ref, loss_ref):
    s = jnp.maximum(s_ref[0:G, :], 1e-9)
    logits = jnp.log(s)
    logits_ref[...] = logits
    lab = lab_ref[...]
    onehot = (lax.broadcasted_iota(jnp.int32, (G, B), 0) == lab
              ).astype(jnp.float32)
    lse = jnp.log(jnp.sum(s, axis=0, keepdims=True))
    picked = jnp.sum(onehot * logits, axis=0, keepdims=True)
    loss_final = jnp.sum(lse - picked).reshape(1, 1) * (1.0 / B)
    L = L_ref[...]
    Lm = jnp.max(L, axis=1, keepdims=True)
    Llse = jnp.log(jnp.sum(jnp.exp(L - Lm), axis=1, keepdims=True)) + Lm
    diag = (lax.broadcasted_iota(jnp.int32, (G, G), 0) ==
            lax.broadcasted_iota(jnp.int32, (G, G), 1)).astype(jnp.float32)
    Ldiag = jnp.sum(L * diag, axis=1, keepdims=True)
    r = Llse - Ldiag
    rexp = _dot0(r, RSEL_ref[...])
    total = jnp.sum(cnt_ref[...], axis=0, keepdims=True)
    route_sum = jnp.sum(total * rexp).reshape(1, 1)
    loss_ref[...] = loss_final + AUX_W * route_sum * (1.0 / (B * T))


def _sc_scan_body(idsT_hbm, Q_hbm, out_hbm, cnt_hbm, ids_v, Q_v, out_v, cnt_v):
    wid = lax.axis_index("s") * 2 + lax.axis_index("c")
    base = wid * BPW
    pltpu.sync_copy(idsT_hbm.at[:, pl.ds(base, BPW)], ids_v)
    pltpu.sync_copy(Q_hbm, Q_v)
    zeros = jnp.zeros((LANES,), jnp.float32)
    ones = jnp.full((LANES,), 1.0, jnp.float32)
    lane = lax.iota(jnp.int32, LANES)
    for v in range(256 // LANES):
        cnt_v[pl.ds(v * LANES, LANES)] = zeros
    for j in range(G, 16):
        for c in range(BPW // LANES):
            out_v[j, pl.ds(c * LANES, LANES)] = zeros
    for c in range(BPW // LANES):
        col = c * LANES
        init = (ones,) * 6 + (zeros,) * 5 + (ones,)

        def step(t, S, col=col):
            ia = ids_v[2 * t, pl.ds(col, LANES)]
            ib = ids_v[2 * t + 1, pl.ds(col, LANES)]
            a16 = ia * 16
            b16 = ib * 16
            plsc.addupdate_scatter(cnt_v, [a16 + lane], ones)
            plsc.addupdate_scatter(cnt_v, [b16 + lane], ones)
            pidx = ia * G + ib
            q = [plsc.load_gather(Q_v, [pidx + 256 * qq]) for qq in range(12)]
            S0, Sre, Sim, S6 = S[0], S[1:6], S[6:11], S[11]
            nre, nim = [], []
            for f in range(5):
                nre.append(Sre[f] * q[f] - Sim[f] * q[5 + f])
                nim.append(Sre[f] * q[5 + f] + Sim[f] * q[f])
            return (S0 * q[11],) + tuple(nre) + tuple(nim) + (S6 * q[10],)

        S = lax.fori_loop(0, T // 2, step, init)
        S0, Sre, Sim, S6 = S[0], S[1:6], S[6:11], S[11]
        inv = 1.0 / G
        for j in range(G):
            acc = S0 * inv + S6 * (((-1.0) ** j) * inv)
            for f in range(1, 6):
                cA = 2.0 * math.cos(2 * math.pi * j * f / G) * inv
                cB = -2.0 * math.sin(2 * math.pi * j * f / G) * inv
                if abs(cA) > 1e-9:
                    acc = acc + Sre[f - 1] * cA
                if abs(cB) > 1e-9:
                    acc = acc + Sim[f - 1] * cB
            out_v[j, pl.ds(col, LANES)] = acc
    pltpu.sync_copy(out_v, out_hbm.at[:, pl.ds(base, BPW)])
    pltpu.sync_copy(cnt_v, cnt_hbm.at[wid])


@functools.cache
def _sc_scan_kernel():
    return pl.kernel(
        _sc_scan_body,
        out_type=[
            jax.ShapeDtypeStruct((16, B), jnp.float32),
            jax.ShapeDtypeStruct((NW, 256), jnp.float32),
        ],
        mesh=plsc.VectorSubcoreMesh(core_axis_name="c", subcore_axis_name="s",
                                    num_cores=2, num_subcores=16),
        scratch_types=[
            pltpu.VMEM((T, BPW), jnp.int32),
            pltpu.VMEM((16 * 256,), jnp.float32),
            pltpu.VMEM((16, BPW), jnp.float32),
            pltpu.VMEM((256,), jnp.float32),
        ],
        compiler_params=pltpu.CompilerParams(needs_layout_passes=False),
    )


def kernel(input_ids, labels, mul, embed, W1, b1, W2, b2):
    del mul
    L, Qp = pl.pallas_call(
        _table_body,
        out_shape=[
            jax.ShapeDtypeStruct((G, G), jnp.float32),
            jax.ShapeDtypeStruct((16, 256), jnp.float32),
        ],
    )(embed, W1, b1, W2, b2,
      jnp.asarray(_CBASIS), jnp.asarray(_E1T), jnp.asarray(_E2T))

    s_final16, cnt = _sc_scan_kernel()(input_ids.T, Qp.reshape(16 * 256))

    logitsT, loss = pl.pallas_call(
        _final_body,
        out_shape=[
            jax.ShapeDtypeStruct((G, B), jnp.float32),
            jax.ShapeDtypeStruct((1, 1), jnp.float32),
        ],
    )(s_final16, labels, cnt, L, jnp.asarray(_RSEL))
    return (logitsT.T, loss.reshape(()))


# speedup vs baseline: 1.2066x; 1.0202x over previous
import functools
import math

import jax
import jax.numpy as jnp
import numpy as np
from jax import lax
from jax.experimental import pallas as pl
from jax.experimental.pallas import tpu as pltpu
from jax.experimental.pallas import tpu_sc as plsc

G = 12
D = 128
T = 50
B = 4096
TEMP = 1.0
AUX_W = 5.0
ID_ID = 0
NW = 32
BPW = B // NW
LANES = 16


def _dot(a, b):
    return lax.dot_general(a, b, (((1,), (0,)), ((), ())),
                           preferred_element_type=jnp.float32,
                           precision=lax.Precision.HIGHEST)


_CB = np.zeros((G, 16), np.float64)
for _g in range(G):
    for _f in range(1, 6):
        _CB[_g, _f - 1] = math.cos(2 * math.pi * _g * _f / G)
        _CB[_g, 4 + _f] = -math.sin(2 * math.pi * _g * _f / G)
    _CB[_g, 10] = (-1.0) ** _g
    _CB[_g, 11] = 1.0
_CBASIS = _CB.astype(np.float32)

_E1T = np.repeat(np.eye(G, dtype=np.float32), G, axis=0).T.copy()
_E2T = np.tile(np.eye(G, dtype=np.float32), (G, 1)).T.copy()
_RSEL = np.zeros((G, 256), np.float32)
for _v in range(G):
    _RSEL[_v, _v * LANES:(_v + 1) * LANES] = 1.0


def _dot0(a, b):
    return lax.dot_general(a, b, (((0,), (0,)), ((), ())),
                           preferred_element_type=jnp.float32,
                           precision=lax.Precision.HIGHEST)


def _table_body(embed_ref, W1_ref, b1_ref, W2_ref, b2_ref, C_ref,
                E1T_ref, E2T_ref, L_ref, Qp_ref):
    z = jnp.maximum(_dot(embed_ref[...], W1_ref[...]) + b1_ref[...], 0.0)
    l = _dot(z, W2_ref[...]) + b2_ref[...]
    L_ref[...] = l
    m = jnp.max(l, axis=1, keepdims=True)
    e = jnp.exp((l - m) * (1.0 / TEMP))
    p = e / jnp.sum(e, axis=1, keepdims=True)
    qT = lax.dot_general(C_ref[...], p, (((0,), (1,)), ((), ())),
                         preferred_element_type=jnp.float32,
                         precision=lax.Precision.HIGHEST)
    qa = _dot(qT, E1T_ref[...])
    qb = _dot(qT, E2T_ref[...])
    are, aim = qa[0:5, :], qa[5:10, :]
    bre, bim = qb[0:5, :], qb[5:10, :]
    body = jnp.concatenate([
        are * bre - aim * bim,
        are * bim + aim * bre,
        qa[10:11, :] * qb[10:11, :],
        qa[11:12, :] * qb[11:12, :],
        jnp.zeros((4, G * G), jnp.float32),
    ], axis=0)
    Qp_ref[...] = jnp.concatenate(
        [body, jnp.zeros((16, 256 - G * G), jnp.float32)], axis=1)


def _final_body(s_ref, lab_ref, cnt_ref, L_ref, RSEL_ref,
                logits_ref, loss_ref):
    s = jnp.maximum(s_ref[0:G, :], 1e-9)
    logits = jnp.log(s)
    logits_ref[...] = logits
    lab = lab_ref[...]
    onehot = (lax.broadcasted_iota(jnp.int32, (G, B), 0) == lab
              ).astype(jnp.float32)
    lse = jnp.log(jnp.sum(s, axis=0, keepdims=True))
    picked = jnp.sum(onehot * logits, axis=0, keepdims=True)
    loss_final = jnp.sum(lse - picked).reshape(1, 1) * (1.0 / B)
    L = L_ref[...]
    Lm = jnp.max(L, axis=1, keepdims=True)
    Llse = jnp.log(jnp.sum(jnp.exp(L - Lm), axis=1, keepdims=True)) + Lm
    diag = (lax.broadcasted_iota(jnp.int32, (G, G), 0) ==
            lax.broadcasted_iota(jnp.int32, (G, G), 1)).astype(jnp.float32)
    Ldiag = jnp.sum(L * diag, axis=1, keepdims=True)
    r = Llse - Ldiag
    rexp = _dot0(r, RSEL_ref[...])
    total = jnp.sum(cnt_ref[...], axis=0, keepdims=True)
    route_sum = jnp.sum(total * rexp).reshape(1, 1)
    loss_ref[...] = loss_final + AUX_W * route_sum * (1.0 / (B * T))


def _sc_scan_body(idsT_hbm, Q_hbm, out_hbm, cnt_hbm, ids_v, Q_v, out_v, cnt_v):
    wid = lax.axis_index("s") * 2 + lax.axis_index("c")
    base = wid * BPW
    pltpu.sync_copy(idsT_hbm.at[:, pl.ds(base, BPW)], ids_v)
    pltpu.sync_copy(Q_hbm, Q_v)
    zeros = jnp.zeros((LANES,), jnp.float32)
    ones = jnp.full((LANES,), 1.0, jnp.float32)
    lane = lax.iota(jnp.int32, LANES)
    for v in range(256 // LANES):
        cnt_v[pl.ds(v * LANES, LANES)] = zeros
    for j in range(G, 16):
        for c in range(BPW // LANES):
            out_v[j, pl.ds(c * LANES, LANES)] = zeros
    def _one(t, S, col):
        ia = ids_v[2 * t, pl.ds(col, LANES)]
        ib = ids_v[2 * t + 1, pl.ds(col, LANES)]
        plsc.addupdate_scatter(cnt_v, [ia * 16 + lane], ones)
        plsc.addupdate_scatter(cnt_v, [ib * 16 + lane], ones)
        pidx = ia * G + ib
        q = [plsc.load_gather(Q_v, [pidx + 256 * qq]) for qq in range(12)]
        S0, Sre, Sim, S6 = S[0], S[1:6], S[6:11], S[11]
        nre, nim = [], []
        for f in range(5):
            nre.append(Sre[f] * q[f] - Sim[f] * q[5 + f])
            nim.append(Sre[f] * q[5 + f] + Sim[f] * q[f])
        return (S0 * q[11],) + tuple(nre) + tuple(nim) + (S6 * q[10],)

    def _emit(S, col):
        S0, Sre, Sim, S6 = S[0], S[1:6], S[6:11], S[11]
        inv = 1.0 / G
        for j in range(G):
            acc = S0 * inv + S6 * (((-1.0) ** j) * inv)
            for f in range(1, 6):
                cA = 2.0 * math.cos(2 * math.pi * j * f / G) * inv
                cB = -2.0 * math.sin(2 * math.pi * j * f / G) * inv
                if abs(cA) > 1e-9:
                    acc = acc + Sre[f - 1] * cA
                if abs(cB) > 1e-9:
                    acc = acc + Sim[f - 1] * cB
            out_v[j, pl.ds(col, LANES)] = acc

    init1 = (ones,) * 6 + (zeros,) * 5 + (ones,)
    for c in range(BPW // LANES // 2):
        colA = (2 * c) * LANES
        colB = (2 * c + 1) * LANES

        def step(t, S, colA=colA, colB=colB):
            SA = _one(t, S[:12], colA)
            SB = _one(t, S[12:], colB)
            return SA + SB

        S = lax.fori_loop(0, T // 2, step, init1 + init1)
        _emit(S[:12], colA)
        _emit(S[12:], colB)
    pltpu.sync_copy(out_v, out_hbm.at[:, pl.ds(base, BPW)])
    pltpu.sync_copy(cnt_v, cnt_hbm.at[wid])


@functools.cache
def _sc_scan_kernel():
    return pl.kernel(
        _sc_scan_body,
        out_type=[
            jax.ShapeDtypeStruct((16, B), jnp.float32),
            jax.ShapeDtypeStruct((NW, 256), jnp.float32),
        ],
        mesh=plsc.VectorSubcoreMesh(core_axis_name="c", subcore_axis_name="s",
                                    num_cores=2, num_subcores=16),
        scratch_types=[
            pltpu.VMEM((T, BPW), jnp.int32),
            pltpu.VMEM((16 * 256,), jnp.float32),
            pltpu.VMEM((16, BPW), jnp.float32),
            pltpu.VMEM((256,), jnp.float32),
        ],
        compiler_params=pltpu.CompilerParams(needs_layout_passes=False),
    )


def kernel(input_ids, labels, mul, embed, W1, b1, W2, b2):
    del mul
    L, Qp = pl.pallas_call(
        _table_body,
        out_shape=[
            jax.ShapeDtypeStruct((G, G), jnp.float32),
            jax.ShapeDtypeStruct((16, 256), jnp.float32),
        ],
    )(embed, W1, b1, W2, b2,
      jnp.asarray(_CBASIS), jnp.asarray(_E1T), jnp.asarray(_E2T))

    s_final16, cnt = _sc_scan_kernel()(input_ids.T, Qp.reshape(16 * 256))

    logitsT, loss = pl.pallas_call(
        _final_body,
        out_shape=[
            jax.ShapeDtypeStruct((G, B), jnp.float32),
            jax.ShapeDtypeStruct((1, 1), jnp.float32),
        ],
    )(s_final16, labels, cnt, L, jnp.asarray(_RSEL))
    return (logitsT.T, loss.reshape(()))


# speedup vs baseline: 1.2548x; 1.0399x over previous
import functools
import math

import jax
import jax.numpy as jnp
import numpy as np
from jax import lax
from jax.experimental import pallas as pl
from jax.experimental.pallas import tpu as pltpu
from jax.experimental.pallas import tpu_sc as plsc

G = 12
D = 128
T = 50
B = 4096
TEMP = 1.0
AUX_W = 5.0
ID_ID = 0
NW = 32
BPW = B // NW
LANES = 16


def _dot(a, b):
    return lax.dot_general(a, b, (((1,), (0,)), ((), ())),
                           preferred_element_type=jnp.float32,
                           precision=lax.Precision.HIGHEST)


_CB = np.zeros((G, 16), np.float64)
for _g in range(G):
    for _f in range(1, 6):
        _CB[_g, _f - 1] = math.cos(2 * math.pi * _g * _f / G)
        _CB[_g, 4 + _f] = -math.sin(2 * math.pi * _g * _f / G)
    _CB[_g, 10] = (-1.0) ** _g
    _CB[_g, 11] = 1.0
_CBASIS = _CB.astype(np.float32)

_E1T = np.repeat(np.eye(G, dtype=np.float32), G, axis=0).T.copy()
_E2T = np.tile(np.eye(G, dtype=np.float32), (G, 1)).T.copy()
_RSEL = np.zeros((G, 256), np.float32)
for _v in range(G):
    _RSEL[_v, _v * LANES:(_v + 1) * LANES] = 1.0


def _dot0(a, b):
    return lax.dot_general(a, b, (((0,), (0,)), ((), ())),
                           preferred_element_type=jnp.float32,
                           precision=lax.Precision.HIGHEST)


def _table_body(embed_ref, W1_ref, b1_ref, W2_ref, b2_ref, C_ref,
                E1T_ref, E2T_ref, L_ref, Qp_ref):
    z = jnp.maximum(_dot(embed_ref[...], W1_ref[...]) + b1_ref[...], 0.0)
    l = _dot(z, W2_ref[...]) + b2_ref[...]
    L_ref[...] = l
    m = jnp.max(l, axis=1, keepdims=True)
    e = jnp.exp((l - m) * (1.0 / TEMP))
    p = e / jnp.sum(e, axis=1, keepdims=True)
    qT = lax.dot_general(C_ref[...], p, (((0,), (1,)), ((), ())),
                         preferred_element_type=jnp.float32,
                         precision=lax.Precision.HIGHEST)
    qa = _dot(qT, E1T_ref[...])
    qb = _dot(qT, E2T_ref[...])
    are, aim = qa[0:5, :], qa[5:10, :]
    bre, bim = qb[0:5, :], qb[5:10, :]
    body = jnp.concatenate([
        are * bre - aim * bim,
        are * bim + aim * bre,
        qa[10:11, :] * qb[10:11, :],
        qa[11:12, :] * qb[11:12, :],
        jnp.zeros((4, G * G), jnp.float32),
    ], axis=0)
    Qp_ref[...] = jnp.concatenate(
        [body, jnp.zeros((16, 256 - G * G), jnp.float32)], axis=1)


def _final_body(s_ref, lab_ref, cnt_ref, L_ref, RSEL_ref,
                logits_ref, loss_ref):
    s = jnp.maximum(s_ref[0:G, :], 1e-9)
    logits = jnp.log(s)
    logits_ref[...] = logits
    lab = lab_ref[...]
    onehot = (lax.broadcasted_iota(jnp.int32, (G, B), 0) == lab
              ).astype(jnp.float32)
    lse = jnp.log(jnp.sum(s, axis=0, keepdims=True))
    picked = jnp.sum(onehot * logits, axis=0, keepdims=True)
    loss_final = jnp.sum(lse - picked).reshape(1, 1) * (1.0 / B)
    L = L_ref[...]
    Lm = jnp.max(L, axis=1, keepdims=True)
    Llse = jnp.log(jnp.sum(jnp.exp(L - Lm), axis=1, keepdims=True)) + Lm
    diag = (lax.broadcasted_iota(jnp.int32, (G, G), 0) ==
            lax.broadcasted_iota(jnp.int32, (G, G), 1)).astype(jnp.float32)
    Ldiag = jnp.sum(L * diag, axis=1, keepdims=True)
    r = Llse - Ldiag
    rexp = _dot0(r, RSEL_ref[...])
    total = jnp.sum(cnt_ref[...], axis=0, keepdims=True)
    route_sum = jnp.sum(total * rexp).reshape(1, 1)
    loss_ref[...] = loss_final + AUX_W * route_sum * (1.0 / (B * T))


def _sc_scan_body(idsT_hbm, Q_hbm, out_hbm, cnt_hbm, ids_v, Q_v, out_v, cnt_v):
    wid = lax.axis_index("s") * 2 + lax.axis_index("c")
    base = wid * BPW
    pltpu.sync_copy(idsT_hbm.at[:, pl.ds(base, BPW)], ids_v)
    pltpu.sync_copy(Q_hbm, Q_v)
    zeros = jnp.zeros((LANES,), jnp.float32)
    ones = jnp.full((LANES,), 1.0, jnp.float32)
    lane = lax.iota(jnp.int32, LANES)
    qrows = [jnp.full((LANES,), qq, jnp.int32) for qq in range(12)]
    for v in range(256 // LANES):
        cnt_v[pl.ds(v * LANES, LANES)] = zeros
    for j in range(G, 16):
        for c in range(BPW // LANES):
            out_v[j, pl.ds(c * LANES, LANES)] = zeros
    def _one(t, S, col):
        ia = ids_v[2 * t, pl.ds(col, LANES)]
        ib = ids_v[2 * t + 1, pl.ds(col, LANES)]
        plsc.addupdate_scatter(cnt_v, [ia * 16 + lane], ones)
        plsc.addupdate_scatter(cnt_v, [ib * 16 + lane], ones)
        pidx = ia * G + ib
        q = [plsc.load_gather(Q_v, [qrows[qq], pidx]) for qq in range(12)]
        S0, Sre, Sim, S6 = S[0], S[1:6], S[6:11], S[11]
        nre, nim = [], []
        for f in range(5):
            nre.append(Sre[f] * q[f] - Sim[f] * q[5 + f])
            nim.append(Sre[f] * q[5 + f] + Sim[f] * q[f])
        return (S0 * q[11],) + tuple(nre) + tuple(nim) + (S6 * q[10],)

    def _emit(S, col):
        S0, Sre, Sim, S6 = S[0], S[1:6], S[6:11], S[11]
        inv = 1.0 / G
        for j in range(G):
            acc = S0 * inv + S6 * (((-1.0) ** j) * inv)
            for f in range(1, 6):
                cA = 2.0 * math.cos(2 * math.pi * j * f / G) * inv
                cB = -2.0 * math.sin(2 * math.pi * j * f / G) * inv
                if abs(cA) > 1e-9:
                    acc = acc + Sre[f - 1] * cA
                if abs(cB) > 1e-9:
                    acc = acc + Sim[f - 1] * cB
            out_v[j, pl.ds(col, LANES)] = acc

    init1 = (ones,) * 6 + (zeros,) * 5 + (ones,)
    for c in range(BPW // LANES // 2):
        colA = (2 * c) * LANES
        colB = (2 * c + 1) * LANES

        def step(t, S, colA=colA, colB=colB):
            SA = _one(t, S[:12], colA)
            SB = _one(t, S[12:], colB)
            return SA + SB

        S = lax.fori_loop(0, T // 2, step, init1 + init1)
        _emit(S[:12], colA)
        _emit(S[12:], colB)
    pltpu.sync_copy(out_v, out_hbm.at[:, pl.ds(base, BPW)])
    pltpu.sync_copy(cnt_v, cnt_hbm.at[wid])


@functools.cache
def _sc_scan_kernel():
    return pl.kernel(
        _sc_scan_body,
        out_type=[
            jax.ShapeDtypeStruct((16, B), jnp.float32),
            jax.ShapeDtypeStruct((NW, 256), jnp.float32),
        ],
        mesh=plsc.VectorSubcoreMesh(core_axis_name="c", subcore_axis_name="s",
                                    num_cores=2, num_subcores=16),
        scratch_types=[
            pltpu.VMEM((T, BPW), jnp.int32),
            pltpu.VMEM((16, 256), jnp.float32),
            pltpu.VMEM((16, BPW), jnp.float32),
            pltpu.VMEM((256,), jnp.float32),
        ],
        compiler_params=pltpu.CompilerParams(needs_layout_passes=False),
    )


def kernel(input_ids, labels, mul, embed, W1, b1, W2, b2):
    del mul
    L, Qp = pl.pallas_call(
        _table_body,
        out_shape=[
            jax.ShapeDtypeStruct((G, G), jnp.float32),
            jax.ShapeDtypeStruct((16, 256), jnp.float32),
        ],
    )(embed, W1, b1, W2, b2,
      jnp.asarray(_CBASIS), jnp.asarray(_E1T), jnp.asarray(_E2T))

    s_final16, cnt = _sc_scan_kernel()(input_ids.T, Qp)

    logitsT, loss = pl.pallas_call(
        _final_body,
        out_shape=[
            jax.ShapeDtypeStruct((G, B), jnp.float32),
            jax.ShapeDtypeStruct((1, 1), jnp.float32),
        ],
    )(s_final16, labels, cnt, L, jnp.asarray(_RSEL))
    return (logitsT.T, loss.reshape(()))
